# baseline (speedup 1.0000x reference)
import functools

import jax
import jax.numpy as jnp
from jax import lax
from jax.experimental import pallas as pl
from jax.experimental.pallas import tpu as pltpu
from jax.experimental.pallas import tpu_sc as plsc


def _argmin_block_kernel(l_ref, rd_ref, idx1_ref, idx2_ref, ckey_ref,
                         *, n_iblocks, ti, n1, n2):
    b = pl.program_id(0)
    i = pl.program_id(1)

    L = l_ref[0]
    Rd = rd_ref[0]

    d6 = jnp.dot(L, Rd, preferred_element_type=jnp.float32)

    bits = lax.bitcast_convert_type(d6, jnp.uint32) & jnp.uint32(0xFFFFF000)
    jlane = lax.broadcasted_iota(jnp.uint32, (ti, n2), 1)
    isub = lax.broadcasted_iota(jnp.uint32, (ti, n2), 0) + jnp.uint32(i * ti)

    krow = lax.bitcast_convert_type(bits | jlane, jnp.float32)
    kcol = lax.bitcast_convert_type(bits | isub, jnp.float32)

    rk = jnp.min(krow, axis=1, keepdims=True)
    j_star = (lax.bitcast_convert_type(rk, jnp.uint32)
              & jnp.uint32(0xFFF)).astype(jnp.int32)
    off1 = (pl.num_programs(0) + b) * n2
    idx1_ref[0] = j_star + off1

    ck = jnp.min(kcol, axis=0, keepdims=True)

    @pl.when(i == 0)
    def _init_cols():
        ckey_ref[...] = ck

    @pl.when(i != 0)
    def _fold_cols():
        ckey_ref[...] = jnp.minimum(ck, ckey_ref[...])

    @pl.when(i == n_iblocks - 1)
    def _emit_idx2():
        i_star = (lax.bitcast_convert_type(ckey_ref[...], jnp.uint32)
                  & jnp.uint32(0xFFF)).astype(jnp.int32)
        idx2_ref[0] = i_star + b * n1


_CHUNK = 512


def _make_sc_tail(n_workers, rows_per_worker, nc):
    mesh = plsc.VectorSubcoreMesh(core_axis_name="c", subcore_axis_name="s")
    groups = rows_per_worker // 16
    n_chunks = rows_per_worker // _CHUNK

    @functools.partial(
        pl.kernel, mesh=mesh,
        out_type=jax.ShapeDtypeStruct((n_workers * 32,), jnp.float32),
        scratch_types=(
            [pltpu.VMEM((_CHUNK,), jnp.int32) for _ in range(n_chunks)]
            + [
                pltpu.VMEM((6 * rows_per_worker,), jnp.float32),
                pltpu.VMEM((6 * rows_per_worker,), jnp.float32),
                pltpu.VMEM((32,), jnp.float32),
                pltpu.SemaphoreType.DMA,
            ]
        ),
    )
    def sc_tail(f0, f1, f2, f3, f4, f5, ii_hbm, out_hbm, *scratch):
        idx_c = scratch[:n_chunks]
        t_v, q_v, ostage_v, sem = scratch[n_chunks:]
        feats = (f0, f1, f2, f3, f4, f5)
        wid = lax.axis_index("s") * nc + lax.axis_index("c")
        base = wid * rows_per_worker
        for c in range(n_chunks):
            pltpu.sync_copy(ii_hbm.at[pl.ds(base + c * _CHUNK, _CHUNK)],
                            idx_c[c])
        copies = []
        for d in range(6):
            for c in range(n_chunks):
                copies.append(pltpu.async_copy(
                    feats[d].at[idx_c[c]],
                    t_v.at[pl.ds(d * rows_per_worker + c * _CHUNK, _CHUNK)],
                    sem))
        for d in range(6):
            pltpu.sync_copy(
                feats[d].at[pl.ds(base, rows_per_worker)],
                q_v.at[pl.ds(d * rows_per_worker, rows_per_worker)])
        for cp in copies:
            cp.wait()

        acc_x = jnp.zeros((16,), jnp.float32)
        acc_n = jnp.zeros((16,), jnp.float32)
        for g in range(groups):
            q = [q_v[pl.ds(d * rows_per_worker + g * 16, 16)]
                 for d in range(6)]
            t = [t_v[pl.ds(d * rows_per_worker + g * 16, 16)]
                 for d in range(6)]
            def _sq(v):
                return v * v
            dx = (_sq(q[0] - t[0]) + _sq(q[1] - t[1]) + _sq(q[2] - t[2]))
            dm = (_sq(q[3] - t[3]) + _sq(q[4] - t[4]) + _sq(q[5] - t[5]))
            dp = (_sq(q[3] + t[3]) + _sq(q[4] + t[4]) + _sq(q[5] + t[5]))
            acc_x = acc_x + dx
            acc_n = acc_n + jnp.minimum(dm, dp)
        ostage_v[pl.ds(0, 16)] = acc_x
        ostage_v[pl.ds(16, 16)] = acc_n
        pltpu.sync_copy(ostage_v, out_hbm.at[pl.ds(wid * 32, 32)])

    return sc_tail


def _normalize(x, eps=1e-12):
    n = jnp.sqrt(jnp.sum(x * x, axis=2, keepdims=True))
    return x / jnp.maximum(n, eps)


@jax.jit
def kernel(xyz1, xyz2, normal_rebuild, normal_gt):
    B, N1, _ = xyz1.shape
    N2 = xyz2.shape[1]

    nr = _normalize(normal_rebuild)
    ng = _normalize(normal_gt)

    sq1 = jnp.sum(xyz1 * xyz1 + nr * nr, axis=2, keepdims=True)
    sq2 = jnp.sum(xyz2 * xyz2 + ng * ng, axis=2, keepdims=True)

    ones1 = jnp.ones((B, N1, 1), jnp.float32)
    L = jnp.concatenate([xyz1, nr, sq1, ones1], axis=2)
    Rd = jnp.concatenate([-2.0 * xyz2, -2.0 * ng, ones1[:, :N2], sq2],
                         axis=2)
    Rd = jnp.transpose(Rd, (0, 2, 1))

    TI = 1024 if N1 % 1024 == 0 else N1
    n_iblocks = N1 // TI

    idx1, idx2 = pl.pallas_call(
        functools.partial(_argmin_block_kernel, n_iblocks=n_iblocks,
                          ti=TI, n1=N1, n2=N2),
        grid=(B, n_iblocks),
        in_specs=[
            pl.BlockSpec((1, TI, 8), lambda b, i: (b, i, 0)),
            pl.BlockSpec((1, 8, N2), lambda b, i: (b, 0, 0)),
        ],
        out_specs=[
            pl.BlockSpec((1, TI, 1), lambda b, i: (b, i, 0)),
            pl.BlockSpec((1, 1, N2), lambda b, i: (b, 0, 0)),
        ],
        out_shape=[
            jax.ShapeDtypeStruct((B, N1, 1), jnp.int32),
            jax.ShapeDtypeStruct((B, 1, N2), jnp.int32),
        ],
        scratch_shapes=[
            pltpu.VMEM((1, N2), jnp.float32),
        ],
    )(L, Rd)

    feat1 = jnp.concatenate([xyz1, nr], axis=2).reshape(B * N1, 6)
    feat2 = jnp.concatenate([xyz2, ng], axis=2).reshape(B * N2, 6)
    PT = jnp.transpose(jnp.concatenate([feat1, feat2], axis=0), (1, 0))
    F = [PT[d] for d in range(6)]
    II = jnp.concatenate([idx1.reshape(B * N1), idx2.reshape(B * N2)])

    info = plsc.get_sparse_core_info()
    NC, NS = info.num_cores, info.num_subcores
    NW = NC * NS
    RPW = (B * (N1 + N2)) // NW

    partials = _make_sc_tail(NW, RPW, NC)(*F, II).reshape(NW, 2, 16)

    inv_count = 1.0 / (B * N1)
    loss_xyz = jnp.sum(partials[:, 0, :]) * inv_count
    loss_nrm = jnp.sum(partials[:, 1, :]) * inv_count
    return (loss_xyz, loss_nrm)


# speedup vs baseline: 1.0038x; 1.0038x over previous
import functools

import jax
import jax.numpy as jnp
from jax import lax
from jax.experimental import pallas as pl
from jax.experimental.pallas import tpu as pltpu
from jax.experimental.pallas import tpu_sc as plsc


def _argmin_block_kernel(l_ref, rd_ref, idx1_ref, idx2_ref, ckey_ref,
                         *, n_iblocks, ti, n1, n2):
    b = pl.program_id(0)
    i = pl.program_id(1)

    L = l_ref[0]
    Rd = rd_ref[0]

    d6 = jnp.dot(L, Rd, preferred_element_type=jnp.float32)

    bits = lax.bitcast_convert_type(d6, jnp.uint32) & jnp.uint32(0xFFFFF000)
    jlane = lax.broadcasted_iota(jnp.uint32, (ti, n2), 1)
    isub = lax.broadcasted_iota(jnp.uint32, (ti, n2), 0) + jnp.uint32(i * ti)

    krow = lax.bitcast_convert_type(bits | jlane, jnp.float32)
    kcol = lax.bitcast_convert_type(bits | isub, jnp.float32)

    rk = jnp.min(krow, axis=1, keepdims=True)
    j_star = (lax.bitcast_convert_type(rk, jnp.uint32)
              & jnp.uint32(0xFFF)).astype(jnp.int32)
    off1 = (pl.num_programs(0) + b) * n2
    idx1_ref[0] = j_star + off1

    ck = jnp.min(kcol, axis=0, keepdims=True)

    @pl.when(i == 0)
    def _init_cols():
        ckey_ref[...] = ck

    @pl.when(i != 0)
    def _fold_cols():
        ckey_ref[...] = jnp.minimum(ck, ckey_ref[...])

    @pl.when(i == n_iblocks - 1)
    def _emit_idx2():
        i_star = (lax.bitcast_convert_type(ckey_ref[...], jnp.uint32)
                  & jnp.uint32(0xFFF)).astype(jnp.int32)
        idx2_ref[0] = i_star + b * n1


_CHUNK = 512


def _make_sc_tail(n_workers, rows_per_worker, nc):
    mesh = plsc.VectorSubcoreMesh(core_axis_name="c", subcore_axis_name="s")
    groups = rows_per_worker // 16
    n_chunks = rows_per_worker // _CHUNK

    @functools.partial(
        pl.kernel, mesh=mesh,
        out_type=jax.ShapeDtypeStruct((n_workers * 32,), jnp.float32),
        scratch_types=(
            [pltpu.VMEM((_CHUNK,), jnp.int32) for _ in range(n_chunks)]
            + [
                pltpu.VMEM((6 * rows_per_worker,), jnp.float32),
                pltpu.VMEM((6 * rows_per_worker,), jnp.float32),
                pltpu.VMEM((32,), jnp.float32),
                pltpu.SemaphoreType.DMA,
            ]
        ),
    )
    def sc_tail(f0, f1, f2, f3, f4, f5, ii_hbm, out_hbm, *scratch):
        idx_c = scratch[:n_chunks]
        t_v, q_v, ostage_v, sem = scratch[n_chunks:]
        feats = (f0, f1, f2, f3, f4, f5)
        wid = lax.axis_index("s") * nc + lax.axis_index("c")
        base = wid * rows_per_worker
        for c in range(n_chunks):
            pltpu.sync_copy(ii_hbm.at[pl.ds(base + c * _CHUNK, _CHUNK)],
                            idx_c[c])
        copies = []
        for d in range(6):
            for c in range(n_chunks):
                copies.append(pltpu.async_copy(
                    feats[d].at[idx_c[c]],
                    t_v.at[pl.ds(d * rows_per_worker + c * _CHUNK, _CHUNK)],
                    sem))
        for d in range(6):
            copies.append(pltpu.async_copy(
                feats[d].at[pl.ds(base, rows_per_worker)],
                q_v.at[pl.ds(d * rows_per_worker, rows_per_worker)], sem))
        for cp in copies:
            cp.wait()

        acc_x = jnp.zeros((16,), jnp.float32)
        acc_n = jnp.zeros((16,), jnp.float32)
        for g in range(groups):
            q = [q_v[pl.ds(d * rows_per_worker + g * 16, 16)]
                 for d in range(6)]
            t = [t_v[pl.ds(d * rows_per_worker + g * 16, 16)]
                 for d in range(6)]
            def _sq(v):
                return v * v
            dx = (_sq(q[0] - t[0]) + _sq(q[1] - t[1]) + _sq(q[2] - t[2]))
            dm = (_sq(q[3] - t[3]) + _sq(q[4] - t[4]) + _sq(q[5] - t[5]))
            dp = (_sq(q[3] + t[3]) + _sq(q[4] + t[4]) + _sq(q[5] + t[5]))
            acc_x = acc_x + dx
            acc_n = acc_n + jnp.minimum(dm, dp)
        ostage_v[pl.ds(0, 16)] = acc_x
        ostage_v[pl.ds(16, 16)] = acc_n
        pltpu.sync_copy(ostage_v, out_hbm.at[pl.ds(wid * 32, 32)])

    return sc_tail


def _normalize(x, eps=1e-12):
    n = jnp.sqrt(jnp.sum(x * x, axis=2, keepdims=True))
    return x / jnp.maximum(n, eps)


@jax.jit
def kernel(xyz1, xyz2, normal_rebuild, normal_gt):
    B, N1, _ = xyz1.shape
    N2 = xyz2.shape[1]

    nr = _normalize(normal_rebuild)
    ng = _normalize(normal_gt)

    sq1 = jnp.sum(xyz1 * xyz1 + nr * nr, axis=2, keepdims=True)
    sq2 = jnp.sum(xyz2 * xyz2 + ng * ng, axis=2, keepdims=True)

    ones1 = jnp.ones((B, N1, 1), jnp.float32)
    L = jnp.concatenate([xyz1, nr, sq1, ones1], axis=2)
    Rd = jnp.concatenate([-2.0 * xyz2, -2.0 * ng, ones1[:, :N2], sq2],
                         axis=2)
    Rd = jnp.transpose(Rd, (0, 2, 1))

    TI = 1024 if N1 % 1024 == 0 else N1
    n_iblocks = N1 // TI

    idx1, idx2 = pl.pallas_call(
        functools.partial(_argmin_block_kernel, n_iblocks=n_iblocks,
                          ti=TI, n1=N1, n2=N2),
        grid=(B, n_iblocks),
        in_specs=[
            pl.BlockSpec((1, TI, 8), lambda b, i: (b, i, 0)),
            pl.BlockSpec((1, 8, N2), lambda b, i: (b, 0, 0)),
        ],
        out_specs=[
            pl.BlockSpec((1, TI, 1), lambda b, i: (b, i, 0)),
            pl.BlockSpec((1, 1, N2), lambda b, i: (b, 0, 0)),
        ],
        out_shape=[
            jax.ShapeDtypeStruct((B, N1, 1), jnp.int32),
            jax.ShapeDtypeStruct((B, 1, N2), jnp.int32),
        ],
        scratch_shapes=[
            pltpu.VMEM((1, N2), jnp.float32),
        ],
    )(L, Rd)

    feat1 = jnp.concatenate([xyz1, nr], axis=2).reshape(B * N1, 6)
    feat2 = jnp.concatenate([xyz2, ng], axis=2).reshape(B * N2, 6)
    PT = jnp.transpose(jnp.concatenate([feat1, feat2], axis=0), (1, 0))
    F = [PT[d] for d in range(6)]
    II = jnp.concatenate([idx1.reshape(B * N1), idx2.reshape(B * N2)])

    info = plsc.get_sparse_core_info()
    NC, NS = info.num_cores, info.num_subcores
    NW = NC * NS
    RPW = (B * (N1 + N2)) // NW

    partials = _make_sc_tail(NW, RPW, NC)(*F, II).reshape(NW, 2, 16)

    inv_count = 1.0 / (B * N1)
    loss_xyz = jnp.sum(partials[:, 0, :]) * inv_count
    loss_nrm = jnp.sum(partials[:, 1, :]) * inv_count
    return (loss_xyz, loss_nrm)
